# baseline (speedup 1.0000x reference)
import jax
import jax.numpy as jnp
from jax import lax
from jax.experimental import pallas as pl
from jax.experimental.pallas import tpu as pltpu

DIMK = 384
DH = 64
NH = 8
P = 4096
PCH = 1024
HPP = 4
EPS = 1e-5
PREC = None
BF = jnp.bfloat16


def _ln_cols(x, g, b):
    m = jnp.mean(x, axis=0, keepdims=True)
    v = jnp.mean((x - m) ** 2, axis=0, keepdims=True)
    return (x - m) * lax.rsqrt(v + EPS) * g + b


def _l2n_rows(x):
    return x / jnp.maximum(jnp.sqrt(jnp.sum(x * x, axis=1, keepdims=True)),
                           1e-12)


def _top16_idxs(s, lt):
    st = s.reshape(1, 64)
    ii = lax.broadcasted_iota(jnp.int32, (64, 64), 0)
    jj = lax.broadcasted_iota(jnp.int32, (64, 64), 1)
    beats = (st > s) | ((st == s) & (jj < ii))
    rank = jnp.sum(beats.astype(jnp.float32), axis=1, keepdims=True)
    sel = rank < 16.0
    slot = jnp.dot(lt, sel.astype(jnp.float32),
                   preferred_element_type=jnp.float32) - 1.0
    iof = lax.broadcasted_iota(jnp.int32, (64, 1), 0)
    return [jnp.sum(jnp.where(sel & (slot == float(t)), iof, 0))
            for t in range(16)]


def _q_kernel(qs_ref, wq_ref, qng_ref, qnb_ref, qn_ref, qp_ref):
    j = pl.program_id(1)
    qsn = _ln_cols(qs_ref[0], qng_ref[...], qnb_ref[...])
    q = lax.dot_general(qsn, wq_ref[...], (((0,), (0,)), ((), ())),
                        preferred_element_type=jnp.float32,
                        precision=PREC)
    for h in range(NH):
        qn_h = _l2n_rows(q[:, h * DH:(h + 1) * DH])
        qn_ref[0, h] = qn_h.astype(BF)
        part = jnp.sum(qn_h, axis=0, keepdims=True)
        @pl.when(j == 0)
        def _():
            qp_ref[0, h] = part[0]
        @pl.when(j != 0)
        def _():
            qp_ref[0, h] = qp_ref[0, h] + part[0]


def _kv_kernel(ctx_ref, wk_ref, wv_ref, cng_ref, cnb_ref, qp_ref,
               kf_ref, vf_ref, k_s, v_s, stk_s, stv_s):
    ctxn = _ln_cols(ctx_ref[0], cng_ref[...], cnb_ref[...])
    k_s[...] = lax.dot_general(ctxn, wk_ref[...], (((0,), (0,)), ((), ())),
                               preferred_element_type=jnp.float32,
                               precision=PREC)
    v_s[...] = lax.dot_general(ctxn.astype(BF), wv_ref[...].astype(BF),
                               (((0,), (0,)), ((), ())),
                               preferred_element_type=jnp.float32,
                               precision=PREC).astype(BF)

    i6 = lax.broadcasted_iota(jnp.int32, (64, 64), 0)
    j6 = lax.broadcasted_iota(jnp.int32, (64, 64), 1)
    lt = (j6 <= i6).astype(jnp.float32)

    for hl in range(HPP):
        c0 = hl * DH
        k_h = k_s[:, c0:c0 + DH]
        kn = _l2n_rows(k_h)
        ka3 = jnp.abs(kn).reshape(64, 64, 64)
        k_height = jnp.sum(ka3, axis=1)
        k_width = jnp.sum(ka3, axis=0)
        qp = qp_ref[0, 0, hl][None, :]
        score_r = jnp.sum(k_height * qp, axis=1, keepdims=True)
        score_c = jnp.sum(qp) * jnp.sum(k_width, axis=1, keepdims=True)

        hs = _top16_idxs(score_r, lt)
        ws = _top16_idxs(score_c, lt)

        for i, hh in enumerate(hs):
            stk_s[i] = k_s[pl.ds(hh * 64, 64), c0:c0 + DH]
            stv_s[i] = v_s[pl.ds(hh * 64, 64), c0:c0 + DH].astype(jnp.float32)
        kf = jnp.concatenate([stk_s[:, pl.ds(w, 1), :] for w in ws],
                             axis=1).reshape(256, DH)
        vf = jnp.concatenate([stv_s[:, pl.ds(w, 1), :] for w in ws],
                             axis=1).reshape(256, DH)
        kf_ref[0, hl] = _l2n_rows(kf).astype(BF)
        vf_ref[0, hl] = vf.astype(BF)


def _attn_kernel(qn_ref, kf_ref, vf_ref, o_ref):
    qn = qn_ref[0, 0]
    kf = kf_ref[0, 0]
    vf = vf_ref[0, 0]
    sim = lax.dot_general(qn, kf, (((1,), (1,)), ((), ())),
                          preferred_element_type=jnp.float32,
                          precision=PREC)
    mx = jnp.max(sim, axis=1, keepdims=True)
    e = jnp.exp(sim - mx)
    o = jnp.dot(e.astype(BF), vf, preferred_element_type=jnp.float32,
                precision=PREC)
    o_ref[0, 0] = (o / jnp.sum(e, axis=1, keepdims=True)).astype(BF)


def _out_kernel(x_ref, wo_ref, og_ref, ob_ref, res_ref, o_ref):
    x = jnp.concatenate([x_ref[0, h] for h in range(NH)], axis=1)
    y = jnp.dot(x, wo_ref[...].astype(BF),
                preferred_element_type=jnp.float32, precision=PREC)
    m = jnp.mean(y, axis=1, keepdims=True)
    v = jnp.mean((y - m) ** 2, axis=1, keepdims=True)
    y = (y - m) * lax.rsqrt(v + EPS) * og_ref[...] + ob_ref[...]
    o_ref[0] = y.T + res_ref[0]


def kernel(query_source, context, W_q, W_kv, W_out, cn_g, cn_b, qn_g, qn_b,
           on_g, on_b, gamma):
    b = query_source.shape[0]
    qs_c = query_source.reshape(b, DIMK, P)
    ctx_c = context.reshape(b, DIMK, P)
    wqT = W_q.T
    wkvT = W_kv.T
    woT = W_out.T
    qng = qn_g.reshape(DIMK, 1)
    qnb = qn_b.reshape(DIMK, 1)
    cng = cn_g.reshape(DIMK, 1)
    cnb = cn_b.reshape(DIMK, 1)
    og = (gamma[0] * on_g).reshape(1, DIMK)
    ob = (gamma[0] * on_b).reshape(1, DIMK)

    qn, qp = pl.pallas_call(
        _q_kernel,
        grid=(b, P // PCH),
        in_specs=[
            pl.BlockSpec((1, DIMK, PCH), lambda i, j: (i, 0, j)),
            pl.BlockSpec((DIMK, 512), lambda i, j: (0, 0)),
            pl.BlockSpec((DIMK, 1), lambda i, j: (0, 0)),
            pl.BlockSpec((DIMK, 1), lambda i, j: (0, 0)),
        ],
        out_specs=[
            pl.BlockSpec((1, NH, PCH, DH), lambda i, j: (i, 0, j, 0)),
            pl.BlockSpec((1, NH, DH), lambda i, j: (i, 0, 0)),
        ],
        out_shape=[
            jax.ShapeDtypeStruct((b, NH, P, DH), BF),
            jax.ShapeDtypeStruct((b, NH, DH), jnp.float32),
        ],
    )(qs_c, wqT, qng, qnb)

    qp4 = qp.reshape(b, NH // HPP, HPP, DH)

    kf, vf = pl.pallas_call(
        _kv_kernel,
        grid=(b, NH // HPP),
        in_specs=[
            pl.BlockSpec((1, DIMK, P), lambda i, j: (i, 0, 0)),
            pl.BlockSpec((DIMK, HPP * DH), lambda i, j: (0, j)),
            pl.BlockSpec((DIMK, HPP * DH), lambda i, j: (0, j + NH // HPP)),
            pl.BlockSpec((DIMK, 1), lambda i, j: (0, 0)),
            pl.BlockSpec((DIMK, 1), lambda i, j: (0, 0)),
            pl.BlockSpec((1, 1, HPP, DH), lambda i, j: (i, j, 0, 0)),
        ],
        out_specs=[
            pl.BlockSpec((1, HPP, 256, DH), lambda i, j: (i, j, 0, 0)),
            pl.BlockSpec((1, HPP, 256, DH), lambda i, j: (i, j, 0, 0)),
        ],
        out_shape=[
            jax.ShapeDtypeStruct((b, NH, 256, DH), BF),
            jax.ShapeDtypeStruct((b, NH, 256, DH), BF),
        ],
        scratch_shapes=[
            pltpu.VMEM((P, HPP * DH), jnp.float32),
            pltpu.VMEM((P, HPP * DH), BF),
            pltpu.VMEM((16, 64, DH), jnp.float32),
            pltpu.VMEM((16, 64, DH), jnp.float32),
        ],
    )(ctx_c, wkvT, wkvT, cng, cnb, qp4)

    attn_out = pl.pallas_call(
        _attn_kernel,
        grid=(b, NH),
        in_specs=[
            pl.BlockSpec((1, 1, P, DH), lambda i, h: (i, h, 0, 0)),
            pl.BlockSpec((1, 1, 256, DH), lambda i, h: (i, h, 0, 0)),
            pl.BlockSpec((1, 1, 256, DH), lambda i, h: (i, h, 0, 0)),
        ],
        out_specs=pl.BlockSpec((1, 1, P, DH), lambda i, h: (i, h, 0, 0)),
        out_shape=jax.ShapeDtypeStruct((b, NH, P, DH), BF),
    )(qn, kf, vf)

    out = pl.pallas_call(
        _out_kernel,
        grid=(b, P // PCH),
        in_specs=[
            pl.BlockSpec((1, NH, PCH, DH), lambda i, j: (i, 0, j, 0)),
            pl.BlockSpec((NH * DH, DIMK), lambda i, j: (0, 0)),
            pl.BlockSpec((1, DIMK), lambda i, j: (0, 0)),
            pl.BlockSpec((1, DIMK), lambda i, j: (0, 0)),
            pl.BlockSpec((1, DIMK, PCH), lambda i, j: (i, 0, j)),
        ],
        out_specs=pl.BlockSpec((1, DIMK, PCH), lambda i, j: (i, 0, j)),
        out_shape=jax.ShapeDtypeStruct((b, DIMK, P), jnp.float32),
    )(attn_out, woT, og, ob, qs_c)

    return out.reshape(b, DIMK, 64, 64)


# speedup vs baseline: 1.1963x; 1.1963x over previous
import jax
import jax.numpy as jnp
from jax import lax
from jax.experimental import pallas as pl
from jax.experimental.pallas import tpu as pltpu

DIMK = 384
DH = 64
NH = 8
P = 4096
PCH = 512
EPS = 1e-5
PREC = None


def _ln_rows(x, g, b):
    m = jnp.mean(x, axis=1, keepdims=True)
    v = jnp.mean((x - m) ** 2, axis=1, keepdims=True)
    return (x - m) * lax.rsqrt(v + EPS) * g + b


def _qkv_kernel(qs_ref, ctx_ref, wq_ref, wkv_ref, qng_ref, qnb_ref,
                cng_ref, cnb_ref, q_ref, k_ref, v_ref):
    qsn = _ln_rows(qs_ref[0], qng_ref[...], qnb_ref[...])
    ctxn = _ln_rows(ctx_ref[0], cng_ref[...], cnb_ref[...])
    q = jnp.dot(qsn, wq_ref[...], preferred_element_type=jnp.float32,
                precision=PREC)
    kv = jnp.dot(ctxn, wkv_ref[...], preferred_element_type=jnp.float32,
                 precision=PREC)
    for h in range(NH):
        q_ref[0, h] = q[:, h * DH:(h + 1) * DH]
    for h in range(NH):
        k_ref[0, h] = kv[:, h * DH:(h + 1) * DH]
        v_ref[0, h] = kv[:, 512 + h * DH:512 + (h + 1) * DH].astype(jnp.bfloat16)


def _top16(s):
    iota = lax.broadcasted_iota(jnp.int32, (64, 1), 0)
    idxs = []
    for _ in range(16):
        m = jnp.max(s)
        idx = jnp.min(jnp.where(s == m, iota, 64))
        idxs.append(idx)
        s = jnp.where(iota == idx, -jnp.inf, s)
    return idxs


def _attn_kernel(q_ref, k_ref, v_ref, o_ref, sc_ref):
    q = q_ref[0, 0]
    k = k_ref[0, 0]
    qn = q / jnp.maximum(jnp.sqrt(jnp.sum(q * q, axis=1, keepdims=True)), 1e-12)
    kn = k / jnp.maximum(jnp.sqrt(jnp.sum(k * k, axis=1, keepdims=True)), 1e-12)

    ka3 = jnp.abs(kn).reshape(64, 64, 64)
    k_height = jnp.sum(ka3, axis=1)
    k_width = jnp.sum(ka3, axis=0)
    qp = jnp.sum(qn, axis=0, keepdims=True)
    score_r = jnp.sum(k_height * qp, axis=1, keepdims=True)
    score_c = jnp.sum(qp) * jnp.sum(k_width, axis=1, keepdims=True)

    hs = _top16(score_r)
    ws = _top16(score_c)

    def gather(src_ref):
        for i, h in enumerate(hs):
            sc_ref[i] = src_ref[0, 0, pl.ds(h * 64, 64), :].astype(jnp.float32)
        cols = [sc_ref[:, pl.ds(w, 1), :] for w in ws]
        return jnp.concatenate(cols, axis=1).reshape(256, 64)

    kf_raw = gather(k_ref)
    kf = kf_raw / jnp.maximum(
        jnp.sqrt(jnp.sum(kf_raw * kf_raw, axis=1, keepdims=True)), 1e-12)
    vf = gather(v_ref)

    sim = lax.dot_general(qn.astype(jnp.bfloat16), kf.astype(jnp.bfloat16),
                          (((1,), (1,)), ((), ())),
                          preferred_element_type=jnp.float32,
                          precision=PREC)
    mx = jnp.max(sim, axis=1, keepdims=True)
    e = jnp.exp(sim - mx)
    o = jnp.dot(e.astype(jnp.bfloat16), vf.astype(jnp.bfloat16),
                preferred_element_type=jnp.float32, precision=PREC)
    o_ref[0, 0] = (o / jnp.sum(e, axis=1, keepdims=True)).astype(jnp.bfloat16)


def _out_kernel(x_ref, w_ref, g_ref, b_ref, res_ref, o_ref):
    x = jnp.concatenate([x_ref[0, h] for h in range(NH)], axis=1)
    y = jnp.dot(x, w_ref[...].astype(jnp.bfloat16),
                preferred_element_type=jnp.float32, precision=PREC)
    m = jnp.mean(y, axis=1, keepdims=True)
    v = jnp.mean((y - m) ** 2, axis=1, keepdims=True)
    o_ref[0] = (y - m) * lax.rsqrt(v + EPS) * g_ref[...] + b_ref[...] + res_ref[0]


def kernel(query_source, context, W_q, W_kv, W_out, cn_g, cn_b, qn_g, qn_b,
           on_g, on_b, gamma):
    b = query_source.shape[0]
    qs_p = query_source.reshape(b, DIMK, P).transpose(0, 2, 1)
    ctx_p = context.reshape(b, DIMK, P).transpose(0, 2, 1)
    wqT = W_q.T
    wkvT = W_kv.T
    woT = W_out.T
    qng = qn_g.reshape(1, DIMK)
    qnb = qn_b.reshape(1, DIMK)
    cng = cn_g.reshape(1, DIMK)
    cnb = cn_b.reshape(1, DIMK)
    og = (gamma[0] * on_g).reshape(1, DIMK)
    ob = (gamma[0] * on_b).reshape(1, DIMK)

    q, k, v = pl.pallas_call(
        _qkv_kernel,
        grid=(b, P // PCH),
        in_specs=[
            pl.BlockSpec((1, PCH, DIMK), lambda i, j: (i, j, 0)),
            pl.BlockSpec((1, PCH, DIMK), lambda i, j: (i, j, 0)),
            pl.BlockSpec((DIMK, 512), lambda i, j: (0, 0)),
            pl.BlockSpec((DIMK, 1024), lambda i, j: (0, 0)),
            pl.BlockSpec((1, DIMK), lambda i, j: (0, 0)),
            pl.BlockSpec((1, DIMK), lambda i, j: (0, 0)),
            pl.BlockSpec((1, DIMK), lambda i, j: (0, 0)),
            pl.BlockSpec((1, DIMK), lambda i, j: (0, 0)),
        ],
        out_specs=[
            pl.BlockSpec((1, NH, PCH, DH), lambda i, j: (i, 0, j, 0)),
            pl.BlockSpec((1, NH, PCH, DH), lambda i, j: (i, 0, j, 0)),
            pl.BlockSpec((1, NH, PCH, DH), lambda i, j: (i, 0, j, 0)),
        ],
        out_shape=[
            jax.ShapeDtypeStruct((b, NH, P, DH), jnp.float32),
            jax.ShapeDtypeStruct((b, NH, P, DH), jnp.float32),
            jax.ShapeDtypeStruct((b, NH, P, DH), jnp.bfloat16),
        ],
    )(qs_p, ctx_p, wqT, wkvT, qng, qnb, cng, cnb)

    attn_out = pl.pallas_call(
        _attn_kernel,
        grid=(b, NH),
        in_specs=[
            pl.BlockSpec((1, 1, P, DH), lambda i, h: (i, h, 0, 0)),
            pl.BlockSpec((1, 1, P, DH), lambda i, h: (i, h, 0, 0)),
            pl.BlockSpec((1, 1, P, DH), lambda i, h: (i, h, 0, 0)),
        ],
        out_specs=pl.BlockSpec((1, 1, P, DH), lambda i, h: (i, h, 0, 0)),
        out_shape=jax.ShapeDtypeStruct((b, NH, P, DH), jnp.bfloat16),
        scratch_shapes=[pltpu.VMEM((16, 64, DH), jnp.float32)],
    )(q, k, v)

    out = pl.pallas_call(
        _out_kernel,
        grid=(b, P // PCH),
        in_specs=[
            pl.BlockSpec((1, NH, PCH, DH), lambda i, j: (i, 0, j, 0)),
            pl.BlockSpec((NH * DH, DIMK), lambda i, j: (0, 0)),
            pl.BlockSpec((1, DIMK), lambda i, j: (0, 0)),
            pl.BlockSpec((1, DIMK), lambda i, j: (0, 0)),
            pl.BlockSpec((1, PCH, DIMK), lambda i, j: (i, j, 0)),
        ],
        out_specs=pl.BlockSpec((1, PCH, DIMK), lambda i, j: (i, j, 0)),
        out_shape=jax.ShapeDtypeStruct((b, P, DIMK), jnp.float32),
    )(attn_out, woT, og, ob, qs_p)

    return out.transpose(0, 2, 1).reshape(b, DIMK, 64, 64)
